# baseline (speedup 1.0000x reference)
import functools

import jax
import jax.numpy as jnp
from jax import lax
from jax.experimental import pallas as pl
from jax.experimental.pallas import tpu as pltpu
from jax.experimental.pallas import tpu_sc as plsc

NUM_ENTITIES = 1000000
NUM_RELATIONS = 1000
EMBED_DIM = 64
BATCH = 16384

NC = 2
NS = 16
LANES = 16
NW = NC * NS

TCOLS = NUM_ENTITIES // 128
TAIL0 = TCOLS * 128
NTAIL = NUM_ENTITIES - TAIL0
LINES = NUM_ENTITIES // 2 + 32
SLABS_PER_W = (TCOLS + NW - 1) // NW

B_PER_W = BATCH // NW
CHUNK = 64
NCHUNK = B_PER_W // CHUNK
GROUPS = CHUNK // LANES
NSLICE = EMBED_DIM // LANES

_MESH = dict(core_axis_name="c", subcore_axis_name="s")


def _transpose_table(ent_t, tail_t):
    @functools.partial(
        pl.kernel,
        out_type=jax.ShapeDtypeStruct((LINES, 128), jnp.float32),
        mesh=plsc.VectorSubcoreMesh(**_MESH),
        compiler_params=pltpu.CompilerParams(needs_layout_passes=False),
        scratch_types=[
            pltpu.VMEM((2, EMBED_DIM, 128), jnp.float32),
            pltpu.VMEM((2, 64, 128), jnp.float32),
            pltpu.VMEM((EMBED_DIM, NTAIL), jnp.float32),
            pltpu.SemaphoreType.DMA,
            pltpu.SemaphoreType.DMA,
            pltpu.SemaphoreType.DMA,
        ],
    )
    def k(ent_hbm, tail_hbm, out_hbm, slab_v, line_v, tail_v,
          rsem, wsem0, wsem1):
        wid = lax.axis_index("s") * NC + lax.axis_index("c")
        lanes = lax.iota(jnp.int32, 16)
        wsems = (wsem0, wsem1)

        lvecs, cvecs = [], []
        for ci in range(8):
            li = ci * LANES + lanes
            lvecs.append(li // 2)
            cvecs.append((li % 2) * 64)

        def read_slab(kk, p):
            tc = kk * NW + wid

            @pl.when(tc < TCOLS)
            def _():
                start = pl.multiple_of(tc * 128, 128)
                pltpu.make_async_copy(
                    ent_hbm.at[:, pl.ds(start, 128)], slab_v.at[p],
                    rsem).start()

        def wait_slab(p):
            pltpu.make_async_copy(
                ent_hbm.at[:, pl.ds(0, 128)], slab_v.at[p], rsem).wait()

        def transpose_slab(p):
            def body(d, _):
                for ci in range(8):
                    v = slab_v[p, d, pl.ds(ci * LANES, LANES)]
                    plsc.store_scatter(
                        line_v.at[p], [lvecs[ci], cvecs[ci] + d], v)
                return 0

            lax.fori_loop(0, EMBED_DIM, body, 0)

        def write_lines(kk, p):
            tc = kk * NW + wid

            @pl.when(tc < TCOLS)
            def _():
                pltpu.make_async_copy(
                    line_v.at[p], out_hbm.at[pl.ds(tc * 64, 64)],
                    wsems[p]).start()

        def drain_lines(p):
            pltpu.make_async_copy(
                line_v.at[p], out_hbm.at[pl.ds(0, 64)], wsems[p]).wait()

        NSTEPS = SLABS_PER_W + (SLABS_PER_W % 2)

        read_slab(0, 0)

        def step_one(kk, p):
            @pl.when(kk + 1 < NSTEPS)
            def _():
                read_slab(kk + 1, 1 - p)

            tc = kk * NW + wid

            @pl.when(tc < TCOLS)
            def _():
                wait_slab(p)

            @pl.when(jnp.logical_and(kk >= 2,
                                     (kk - 2) * NW + wid < TCOLS))
            def _():
                drain_lines(p)

            @pl.when(tc < TCOLS)
            def _():
                transpose_slab(p)
                write_lines(kk, p)

        def step(itr, _):
            step_one(itr * 2, 0)
            step_one(itr * 2 + 1, 1)
            return 0

        lax.fori_loop(0, NSTEPS // 2, step, 0)

        @pl.when((NSTEPS - 2) * NW + wid < TCOLS)
        def _():
            drain_lines(0)

        @pl.when((NSTEPS - 1) * NW + wid < TCOLS)
        def _():
            drain_lines(1)

        @pl.when(wid == 0)
        def _():
            pltpu.sync_copy(tail_hbm, tail_v)

            def body(d, _):
                for ci in range(NTAIL // LANES):
                    v = tail_v[d, pl.ds(ci * LANES, LANES)]
                    plsc.store_scatter(
                        line_v.at[0], [lvecs[ci], cvecs[ci] + d], v)
                return 0

            lax.fori_loop(0, EMBED_DIM, body, 0)
            pltpu.sync_copy(
                line_v.at[0, pl.ds(0, NTAIL // 2)],
                out_hbm.at[pl.ds(TCOLS * 64, NTAIL // 2)])

    return k(ent_t, tail_t)


def _gather_score(heads, relations, tails, ent_lin, relw):
    @functools.partial(
        pl.kernel,
        out_type=jax.ShapeDtypeStruct((BATCH,), jnp.float32),
        mesh=plsc.VectorSubcoreMesh(**_MESH),
        compiler_params=pltpu.CompilerParams(needs_layout_passes=False),
        scratch_types=[
            pltpu.VMEM((B_PER_W,), jnp.int32),
            pltpu.VMEM((B_PER_W,), jnp.int32),
            pltpu.VMEM((B_PER_W,), jnp.int32),
            pltpu.VMEM((NUM_RELATIONS // 2, 128), jnp.float32),
            pltpu.VMEM((2, CHUNK), jnp.int32),
            pltpu.VMEM((2, CHUNK), jnp.int32),
            pltpu.VMEM((2, CHUNK, 128), jnp.float32),
            pltpu.VMEM((2, CHUNK, 128), jnp.float32),
            pltpu.VMEM((B_PER_W,), jnp.float32),
            pltpu.VMEM((LANES * LANES,), jnp.float32),
            pltpu.SemaphoreType.DMA,
            pltpu.SemaphoreType.DMA,
        ],
    )
    def k(heads_hbm, rels_hbm, tails_hbm, ent_hbm, relw_hbm, out_hbm,
          hidx_v, tidx_v, ridx_v, relw_v, hl_v, tl_v, h_v, t_v,
          out_v, pt_v, sem0, sem1):
        wid = lax.axis_index("s") * NC + lax.axis_index("c")
        base = wid * B_PER_W

        pltpu.sync_copy(heads_hbm.at[pl.ds(base, B_PER_W)], hidx_v)
        pltpu.sync_copy(tails_hbm.at[pl.ds(base, B_PER_W)], tidx_v)
        pltpu.sync_copy(rels_hbm.at[pl.ds(base, B_PER_W)], ridx_v)
        pltpu.sync_copy(relw_hbm, relw_v)

        sems = (sem0, sem1)
        lanes = lax.iota(jnp.int32, 16)

        def build_fire(c):
            p = c % 2
            sem = sems[p]

            def bld(it, _):
                sl16 = pl.ds(it * LANES, LANES)
                src = pl.ds(c * CHUNK + it * LANES, LANES)
                hl_v[p, sl16] = jax.lax.shift_right_logical(hidx_v[src], 1)
                tl_v[p, sl16] = jax.lax.shift_right_logical(tidx_v[src], 1)
                return 0

            lax.fori_loop(0, CHUNK // LANES, bld, 0)
            pltpu.make_async_copy(
                ent_hbm.at[hl_v.at[p]], h_v.at[p], sem).start()
            pltpu.make_async_copy(
                ent_hbm.at[tl_v.at[p]], t_v.at[p], sem).start()

        def drain_chunk(c):
            p = c % 2
            sem = sems[p]
            pltpu.make_async_copy(
                ent_hbm.at[pl.ds(0, CHUNK)], h_v.at[p], sem).wait()
            pltpu.make_async_copy(
                ent_hbm.at[pl.ds(0, CHUNK)], t_v.at[p], sem).wait()

        def compute_chunk(c):
            p = c % 2

            def group_body(g, _):
                off = c * CHUNK + g * LANES
                rvec = ridx_v[pl.ds(off, LANES)]
                hvec = hidx_v[pl.ds(off, LANES)]
                tvec = tidx_v[pl.ds(off, LANES)]
                for j in range(LANES):
                    i = g * LANES + j
                    qs = rvec[j]
                    ql = jnp.full((16,), qs >> 1, jnp.int32)
                    qoff = (qs & 1) * 64
                    hoff = (hvec[j] & 1) * 64
                    toff = (tvec[j] & 1) * 64
                    terms = []
                    for s in range(NSLICE):
                        dvec = s * LANES + lanes
                        rv = plsc.load_gather(relw_v, [ql, qoff + dvec])
                        hv = h_v[p, i, pl.ds(hoff + s * LANES, LANES)]
                        tv = t_v[p, i, pl.ds(toff + s * LANES, LANES)]
                        terms.append(jnp.abs(hv + rv - tv))
                    part = (terms[0] + terms[1]) + (terms[2] + terms[3])
                    plsc.store_scatter(pt_v, [lanes * LANES + j], part)
                cols = [pt_v[pl.ds(l * LANES, LANES)] for l in range(LANES)]
                while len(cols) > 1:
                    cols = [cols[2 * m] + cols[2 * m + 1]
                            for m in range(len(cols) // 2)]
                out_v[pl.ds(c * CHUNK + g * LANES, LANES)] = cols[0]
                return 0

            lax.fori_loop(0, GROUPS, group_body, 0)

        build_fire(0)
        for c in range(NCHUNK):
            if c + 1 < NCHUNK:
                build_fire(c + 1)
            drain_chunk(c)
            compute_chunk(c)

        pltpu.sync_copy(out_v, out_hbm.at[pl.ds(base, B_PER_W)])

    return k(heads, relations, tails, ent_lin, relw)


def kernel(heads, relations, tails, entity_weight, relation_weight):
    ent_t = entity_weight.T
    tail_t = entity_weight[TAIL0:, :].T
    ent_lin = _transpose_table(ent_t, tail_t)
    relw_lines = relation_weight.reshape(NUM_RELATIONS // 2, 128)
    return _gather_score(heads.astype(jnp.int32), relations.astype(jnp.int32),
                         tails.astype(jnp.int32), ent_lin, relw_lines)
